# baseline (speedup 1.0000x reference)
import jax
import jax.numpy as jnp
from jax.experimental import pallas as pl
from jax.experimental.pallas import tpu as pltpu

CH = 3
FEAT = 16
IMG = 64
SLOTS = 1000
READ = 3
BATCH = 8


def _conv2d(x, w, stride, pad):
    return jax.lax.conv_general_dilated(
        x, w, (stride, stride), [(pad, pad), (pad, pad)],
        dimension_numbers=('NCHW', 'OIHW', 'NCHW'))


def _batchnorm(x, eps=1e-5):
    mean = x.mean(axis=(0, 2, 3), keepdims=True)
    var = x.var(axis=(0, 2, 3), keepdims=True)
    return (x - mean) / jnp.sqrt(var + eps)


def _fast_att(x, w1, b1, w2, b2):
    y = jax.nn.softmax(x @ w1.T + b1, axis=1)
    y = y @ w2.T + b2
    return x * y


def _heads(inputs, p):
    B = inputs.shape[0]
    h = jax.nn.relu(_batchnorm(_conv2d(inputs, p['enc1'], 4, 1)))
    h = jax.nn.relu(_batchnorm(_conv2d(h, p['enc2'], 2, 1)))
    h = jax.nn.relu(_batchnorm(_conv2d(h, p['enc3'], 2, 1)))
    h = jax.nn.relu(_batchnorm(_conv2d(h, p['enc4'], 1, 0)))
    encoded = h.reshape(B, -1)
    rfa = _fast_att(encoded, p['rfa_w1'], p['rfa_b1'], p['rfa_w2'], p['rfa_b2'])
    read = jnp.tanh(rfa @ p['r_w'].T + p['r_b']).reshape(B, READ, SLOTS)
    read_w = jnp.max(read, axis=2)
    read_idx = jnp.argmax(read, axis=2).astype(jnp.int32)
    ufa = _fast_att(encoded, p['ufa_w1'], p['ufa_b1'], p['ufa_w2'], p['ufa_b2'])
    uw = jax.nn.sigmoid(ufa @ p['u_w'].T + p['u_b'])
    return read_idx, read_w, uw


def _conv3x3(x, w_ref):
    ci, h, wd = x.shape
    co = w_ref.shape[0]
    zc = jnp.zeros((ci, h, 1), x.dtype)
    xw = jnp.concatenate([zc, x, zc], axis=2)
    zr = jnp.zeros((ci, 1, wd + 2), x.dtype)
    xp = jnp.concatenate([zr, xw, zr], axis=1)
    outs = []
    for o in range(co):
        acc = None
        for i in range(ci):
            for ky in range(3):
                for kx in range(3):
                    t = w_ref[o, i, ky, kx] * xp[i, ky:ky + h, kx:kx + wd]
                    acc = t if acc is None else acc + t
        outs.append(acc[None])
    return jnp.concatenate(outs, axis=0)


def _softmax_h(x):
    m = jnp.max(x, axis=1, keepdims=True)
    e = jnp.exp(x - m)
    return e / jnp.sum(e, axis=1, keepdims=True)


def _fast_att_img_k(x, w1_ref, w2_ref):
    y = _conv3x3(x, w1_ref)
    y = _softmax_h(y)
    y = _conv3x3(y, w2_ref)
    return x * y


def _update_body(idx_ref, r_ref, inp_ref, uw_ref, rw_ref,
                 um1_ref, um2_ref, um3_ref, am1_ref, am2_ref, am3_ref,
                 out_ref, m_ref):
    s = pl.program_id(0)
    b = pl.program_id(1)

    @pl.when(s == 0)
    def _():
        m_ref[b] = jnp.zeros((CH, IMG, IMG), jnp.float32)

    r = r_ref[0, 0]
    inp = inp_ref[0]

    x = jnp.concatenate([r, inp], axis=0)
    um = _fast_att_img_k(x, um1_ref, um2_ref)
    um = jax.nn.relu(_conv3x3(um, um3_ref))
    w = uw_ref[s, b]
    r2 = w * um + (1.0 - w) * r

    x2 = jnp.concatenate([r2, m_ref[b]], axis=0)
    am = _fast_att_img_k(x2, am1_ref, am2_ref)
    am = jax.nn.relu(_conv3x3(am, am3_ref))
    m = rw_ref[s, b] * am
    m_ref[b] = m
    out_ref[0] = jnp.tanh(m)


def _update_pallas(memory, idx_flat, inputs, uw, read_w, p, interpret=False):
    uw2 = jnp.transpose(uw)
    rw2 = jnp.transpose(read_w)
    smem = pl.BlockSpec(memory_space=pltpu.SMEM)
    grid_spec = pltpu.PrefetchScalarGridSpec(
        num_scalar_prefetch=1,
        grid=(READ, BATCH),
        in_specs=[
            pl.BlockSpec((1, 1, CH, IMG, IMG),
                         lambda s, b, idx: (b, idx[s * BATCH + b], 0, 0, 0)),
            pl.BlockSpec((1, CH, IMG, IMG), lambda s, b, idx: (b, 0, 0, 0)),
            smem, smem, smem, smem, smem, smem, smem, smem,
        ],
        out_specs=pl.BlockSpec((1, CH, IMG, IMG),
                               lambda s, b, idx: (b, 0, 0, 0)),
        scratch_shapes=[
            pltpu.VMEM((BATCH, CH, IMG, IMG), jnp.float32),
        ],
    )
    return pl.pallas_call(
        _update_body,
        grid_spec=grid_spec,
        out_shape=jax.ShapeDtypeStruct((BATCH, CH, IMG, IMG), jnp.float32),
        interpret=interpret,
    )(idx_flat, memory, inputs, uw2, rw2,
      p['um1'], p['um2'], p['um3'], p['am1'], p['am2'], p['am3'])


def kernel(inputs, memory, params):
    read_idx, read_w, uw = _heads(inputs, params)
    idx_flat = jnp.transpose(read_idx).reshape(READ * BATCH)
    return _update_pallas(memory, idx_flat, inputs, uw, read_w, params)


# speedup vs baseline: 2.0575x; 2.0575x over previous
import functools

import jax
import jax.numpy as jnp
from jax import lax
from jax.experimental import pallas as pl
from jax.experimental.pallas import tpu as pltpu
from jax.experimental.pallas import tpu_sc as plsc

CH = 3
FEAT = 16
IMG = 64
SLOTS = 1000
READ = 3
BATCH = 8


def _conv2d(x, w, stride, pad):
    return jax.lax.conv_general_dilated(
        x, w, (stride, stride), [(pad, pad), (pad, pad)],
        dimension_numbers=('NCHW', 'OIHW', 'NCHW'))


def _batchnorm(x, eps=1e-5):
    mean = x.mean(axis=(0, 2, 3), keepdims=True)
    var = x.var(axis=(0, 2, 3), keepdims=True)
    return (x - mean) / jnp.sqrt(var + eps)


def _fast_att(x, w1, b1, w2, b2):
    y = jax.nn.softmax(x @ w1.T + b1, axis=1)
    y = y @ w2.T + b2
    return x * y


def _heads(inputs, p):
    B = inputs.shape[0]
    h = jax.nn.relu(_batchnorm(_conv2d(inputs, p['enc1'], 4, 1)))
    h = jax.nn.relu(_batchnorm(_conv2d(h, p['enc2'], 2, 1)))
    h = jax.nn.relu(_batchnorm(_conv2d(h, p['enc3'], 2, 1)))
    h = jax.nn.relu(_batchnorm(_conv2d(h, p['enc4'], 1, 0)))
    encoded = h.reshape(B, -1)
    rfa = _fast_att(encoded, p['rfa_w1'], p['rfa_b1'], p['rfa_w2'], p['rfa_b2'])
    read = jnp.tanh(rfa @ p['r_w'].T + p['r_b']).reshape(B, READ, SLOTS)
    read_w = jnp.max(read, axis=2)
    read_idx = jnp.argmax(read, axis=2).astype(jnp.int32)
    ufa = _fast_att(encoded, p['ufa_w1'], p['ufa_b1'], p['ufa_w2'], p['ufa_b2'])
    uw = jax.nn.sigmoid(ufa @ p['u_w'].T + p['u_b'])
    return read_idx, read_w, uw


def _sc_gather(memory, read_idx):
    n_per_core = (BATCH * READ) // 2

    @functools.partial(
        pl.kernel,
        out_type=jax.ShapeDtypeStruct((READ, BATCH, CH, IMG, IMG),
                                      jnp.float32),
        mesh=plsc.ScalarSubcoreMesh(axis_name='c', num_cores=2),
        scratch_types=[
            pltpu.SMEM((BATCH, READ), jnp.int32),
            pltpu.SemaphoreType.DMA,
            pltpu.SemaphoreType.DMA,
        ],
        compiler_params=pltpu.CompilerParams(use_tc_tiling_on_sc=True),
    )
    def gather_kernel(mem_hbm, idx_hbm, out_hbm, idx_s, sem_i, sem_d):
        core = lax.axis_index('c')
        pltpu.async_copy(idx_hbm, idx_s, sem_i).wait()

        @pl.loop(0, n_per_core)
        def _issue(i):
            j = core * n_per_core + i
            s = j // BATCH
            b = j % BATCH
            slot = idx_s[b, s]
            pltpu.async_copy(mem_hbm.at[b, slot], out_hbm.at[s, b], sem_d)

        @pl.loop(0, n_per_core)
        def _drain(i):
            pltpu.make_async_copy(mem_hbm.at[0, 0], out_hbm.at[0, 0],
                                  sem_d).wait()

    return gather_kernel(memory, read_idx)


def _shifts9(x):
    z1 = jnp.zeros_like(x[..., :1])
    zrow = jnp.zeros_like(x[..., :1, :])
    out = {}
    for dy in (-1, 0, 1):
        if dy == 0:
            base = x
        elif dy == 1:
            up = jnp.concatenate([x[..., 1:, :], zrow], axis=-2)
            base = jnp.concatenate([x[..., 64:], up[..., :64]], axis=-1)
        else:
            dn = jnp.concatenate([zrow, x[..., :-1, :]], axis=-2)
            base = jnp.concatenate([dn[..., 64:], x[..., :64]], axis=-1)
        for dx in (-1, 0, 1):
            if dx == 0:
                t = base
            elif dx == 1:
                t = jnp.concatenate(
                    [base[..., 1:64], z1, base[..., 65:128], z1], axis=-1)
            else:
                t = jnp.concatenate(
                    [z1, base[..., 0:63], z1, base[..., 64:127]], axis=-1)
            out[(dy, dx)] = t
    return out


def _conv3x3(x, w):
    co, ci = w.shape[0], w.shape[1]
    sh = _shifts9(x)
    outs = []
    for o in range(co):
        acc = None
        for i in range(ci):
            for ky in range(3):
                for kx in range(3):
                    t = w[o, i, ky, kx] * sh[(ky - 1, kx - 1)][:, i]
                    acc = t if acc is None else acc + t
        outs.append(acc[:, None])
    return jnp.concatenate(outs, axis=1)


def _softmax_h(x):
    m1 = jnp.max(x, axis=-2, keepdims=True)
    m64 = jnp.maximum(m1[..., :64], m1[..., 64:])
    e = jnp.exp(x - jnp.concatenate([m64, m64], axis=-1))
    s1 = jnp.sum(e, axis=-2, keepdims=True)
    s64 = s1[..., :64] + s1[..., 64:]
    return e / jnp.concatenate([s64, s64], axis=-1)


def _fast_att_img_k(x, w1, w2):
    y = _conv3x3(x, w1)
    y = _softmax_h(y)
    y = _conv3x3(y, w2)
    return x * y


def _slot_copies(mem_ref, idx_ref, rbuf_ref, sem_ref, g, slot):
    return [
        pltpu.make_async_copy(
            mem_ref.at[b, idx_ref[g * BATCH + b]],
            rbuf_ref.at[slot, b],
            sem_ref.at[slot],
        )
        for b in range(BATCH)
    ]


def _update_body(idx_ref, mem_ref, inp_ref, uw_ref, rw_ref,
                 um1_ref, um2_ref, um3_ref, am1_ref, am2_ref, am3_ref,
                 out_ref, m_ref, rbuf_ref, sem_ref):
    s = pl.program_id(0)

    @pl.when(s == 0)
    def _():
        m_ref[...] = jnp.zeros((BATCH, CH, IMG // 2, 2 * IMG), jnp.float32)
        for c in _slot_copies(mem_ref, idx_ref, rbuf_ref, sem_ref, 0, 0):
            c.start()

    @pl.when(s < READ - 1)
    def _():
        for c in _slot_copies(mem_ref, idx_ref, rbuf_ref, sem_ref,
                              s + 1, (s + 1) % 2):
            c.start()

    for c in _slot_copies(mem_ref, idx_ref, rbuf_ref, sem_ref, s, s % 2):
        c.wait()

    r = rbuf_ref[s % 2]
    inp = inp_ref[...]
    um1, um2, um3 = um1_ref, um2_ref, um3_ref
    am1, am2, am3 = am1_ref, am2_ref, am3_ref

    x = jnp.concatenate([r, inp], axis=1)
    um = _fast_att_img_k(x, um1, um2)
    um = jax.nn.relu(_conv3x3(um, um3))
    w = uw_ref[...].reshape(BATCH, 1, 1, 1)
    r2 = w * um + (1.0 - w) * r

    x2 = jnp.concatenate([r2, m_ref[...]], axis=1)
    am = _fast_att_img_k(x2, am1, am2)
    am = jax.nn.relu(_conv3x3(am, am3))
    m = rw_ref[...].reshape(BATCH, 1, 1, 1) * am
    m_ref[...] = m
    out_ref[...] = jnp.tanh(m)


def _update_pallas(memory, idx_flat, inputs, uw, read_w, p, interpret=False):
    uw3 = jnp.transpose(uw).reshape(READ, BATCH, 1)
    rw3 = jnp.transpose(read_w).reshape(READ, BATCH, 1)
    full4 = pl.BlockSpec((BATCH, CH, IMG // 2, 2 * IMG),
                         lambda s, *_: (0, 0, 0, 0))
    scal = pl.BlockSpec((1, BATCH, 1), lambda s, *_: (s, 0, 0))
    w66 = pl.BlockSpec(memory_space=pltpu.SMEM)
    w36 = pl.BlockSpec(memory_space=pltpu.SMEM)
    grid_spec = pltpu.PrefetchScalarGridSpec(
        num_scalar_prefetch=1,
        grid=(READ,),
        in_specs=[pl.BlockSpec(memory_space=pl.ANY),
                  full4, scal, scal, w66, w66, w36, w66, w66, w36],
        out_specs=full4,
        scratch_shapes=[
            pltpu.VMEM((BATCH, CH, IMG // 2, 2 * IMG), jnp.float32),
            pltpu.VMEM((2, BATCH, CH, IMG // 2, 2 * IMG), jnp.float32),
            pltpu.SemaphoreType.DMA((2,)),
        ],
    )
    return pl.pallas_call(
        _update_body,
        grid_spec=grid_spec,
        out_shape=jax.ShapeDtypeStruct((BATCH, CH, IMG // 2, 2 * IMG),
                                       jnp.float32),
        interpret=interpret,
    )(idx_flat, memory, inputs, uw3, rw3,
      p['um1'], p['um2'], p['um3'], p['am1'], p['am2'], p['am3'])


def kernel(inputs, memory, params):
    read_idx, read_w, uw = _heads(inputs, params)
    idx_flat = jnp.transpose(read_idx).reshape(READ * BATCH)
    mem_v = memory.reshape(BATCH, SLOTS, CH, IMG // 2, 2 * IMG)
    inp_v = inputs.reshape(BATCH, CH, IMG // 2, 2 * IMG)
    out = _update_pallas(mem_v, idx_flat, inp_v, uw, read_w, params)
    return out.reshape(BATCH, CH, IMG, IMG)
